# Initial kernel scaffold; baseline (speedup 1.0000x reference)
import jax
import jax.numpy as jnp
from jax.experimental import pallas as pl


def kernel(hidden_states, ln1_w, wq, wk, wv, q_norm_w, k_norm_w, wo, ln2_w, router_w, w_gate_up, w_down):
    raise NotImplementedError("write your pallas kernel here")


# speedup vs baseline: 1.2193x; 1.2193x over previous
import functools
import math

import jax
import jax.numpy as jnp
from jax.experimental import pallas as pl
from jax.experimental.pallas import tpu as pltpu

D = 1024
H = 16
HK = 4
DH = 64
E = 16
F = 768
B = 32
S = 4
T = B * S
EPS = 1e-06
THETA = 1000000.0
TILE = 16
TP = 256


def _rms(x, w, eps=EPS):
    var = jnp.mean(x * x, axis=-1, keepdims=True)
    return x * jax.lax.rsqrt(var + eps) * w


def _attn_router_body(h_ref, ln1_ref, wq_ref, wk_ref, wv_ref, qn_ref, kn_ref,
                      wo_ref, ln2_ref, rw_ref, cos_ref, sin_ref,
                      hid_ref, xs_ref, p_ref, off_ref, cnt_ref):
    h = h_ref[...]
    x = _rms(h, ln1_ref[...])
    q2 = jnp.dot(x, wq_ref[...], preferred_element_type=jnp.float32)
    k2 = jnp.dot(x, wk_ref[...], preferred_element_type=jnp.float32)
    v2 = jnp.dot(x, wv_ref[...], preferred_element_type=jnp.float32)
    cos = cos_ref[...]
    sin = sin_ref[...]
    qn = qn_ref[...]
    kn = kn_ref[...]

    def rope(z):
        zr = jnp.concatenate([-z[:, DH // 2:], z[:, :DH // 2]], axis=1)
        return z * cos + zr * sin

    ks = []
    vs = []
    for j in range(HK):
        kj = rope(_rms(k2[:, j * DH:(j + 1) * DH], kn))
        ks.append(kj)
        vs.append(v2[:, j * DH:(j + 1) * DH])

    ti = jax.lax.broadcasted_iota(jnp.int32, (T, T), 0)
    tj = jax.lax.broadcasted_iota(jnp.int32, (T, T), 1)
    mask = (ti // S == tj // S) & (tj % S <= ti % S)
    neg = jnp.float32(-1e30)
    scale = jnp.float32(1.0 / math.sqrt(DH))

    outs = []
    for hh in range(H):
        j = hh // (H // HK)
        qh = rope(_rms(q2[:, hh * DH:(hh + 1) * DH], qn))
        sc = jax.lax.dot_general(qh, ks[j], (((1,), (1,)), ((), ())),
                                 preferred_element_type=jnp.float32) * scale
        sc = jnp.where(mask, sc, neg)
        sc = sc - jnp.max(sc, axis=1, keepdims=True)
        ex = jnp.exp(sc)
        at = ex / jnp.sum(ex, axis=1, keepdims=True)
        outs.append(jnp.dot(at, vs[j], preferred_element_type=jnp.float32))
    o2 = jnp.concatenate(outs, axis=1)
    hidden = h + jnp.dot(o2, wo_ref[...], preferred_element_type=jnp.float32)
    hid_ref[...] = hidden

    x2 = _rms(hidden, ln2_ref[...])
    logits = jnp.dot(x2, rw_ref[...], preferred_element_type=jnp.float32)

    eidx = jax.lax.broadcasted_iota(jnp.int32, (T, E), 1)
    rowmax = jnp.max(logits, axis=1, keepdims=True)
    assign = jnp.min(jnp.where(logits == rowmax, eidx, E), axis=1,
                     keepdims=True)
    onehot = (eidx == assign).astype(jnp.float32)
    counts = jnp.sum(onehot, axis=0, keepdims=True)
    padded = jnp.float32(8.0) * jnp.ceil(counts * jnp.float32(1.0 / 8.0))
    er = jax.lax.broadcasted_iota(jnp.int32, (E, E), 0)
    ec = jax.lax.broadcasted_iota(jnp.int32, (E, E), 1)
    lt_e = (er < ec).astype(jnp.float32)
    offsets = jnp.dot(padded, lt_e, preferred_element_type=jnp.float32)
    lt_t = (tj < ti).astype(jnp.float32)
    intra = jnp.dot(lt_t, onehot, preferred_element_type=jnp.float32)
    rank = jnp.sum(onehot * (offsets + intra), axis=1, keepdims=True)
    rj = jax.lax.broadcasted_iota(jnp.int32, (T, TP), 1)
    p_mat = (rank.astype(jnp.int32) == rj).astype(jnp.float32)
    p_ref[...] = p_mat
    xs_ref[...] = jnp.dot(p_mat.T, x2, preferred_element_type=jnp.float32)
    off_ref[...] = (offsets * jnp.float32(1.0 / 8.0)).astype(jnp.int32)
    cnt_ref[...] = counts.astype(jnp.int32)


def _moe_body(off8_sm, cnt_sm, xs_ref, wgu_ref, wd_ref, y_ref):
    e = pl.program_id(0)

    @pl.when(e == 0)
    def _init():
        y_ref[...] = jnp.zeros_like(y_ref)

    off8 = off8_sm[e]
    cnt = cnt_sm[e]
    ntiles = (cnt + TILE - 1) // TILE
    wgu = wgu_ref[0]
    wd = wd_ref[0]
    rid = jax.lax.broadcasted_iota(jnp.int32, (TILE, 1), 0)

    def tile_step(t, _):
        base = off8 * 8 + t * TILE
        xb = xs_ref[pl.ds(base, TILE), :]
        gu = jnp.dot(xb, wgu, preferred_element_type=jnp.float32)
        gate = gu[:, :F]
        up = gu[:, F:]
        act = gate * jax.lax.logistic(gate) * up
        y = jnp.dot(act, wd, preferred_element_type=jnp.float32)
        valid = t * TILE + rid < cnt
        cur = y_ref[pl.ds(base, TILE), :]
        y_ref[pl.ds(base, TILE), :] = jnp.where(valid, y, cur)
        return 0

    jax.lax.fori_loop(0, ntiles, tile_step, 0)


def _unsort_body(hid_ref, p_ref, y_ref, out_ref):
    out_ref[...] = hid_ref[...] + jnp.dot(
        p_ref[...], y_ref[...], preferred_element_type=jnp.float32)


@jax.jit
def kernel(hidden_states, ln1_w, wq, wk, wv, q_norm_w, k_norm_w, wo, ln2_w,
           router_w, w_gate_up, w_down):
    h2 = hidden_states.reshape(T, D)
    pos = jnp.arange(S, dtype=jnp.float32)
    inv = 1.0 / (THETA ** (jnp.arange(0, DH, 2, dtype=jnp.float32) / DH))
    freqs = pos[:, None] * inv[None, :]
    emb = jnp.concatenate([freqs, freqs], axis=-1)
    cos_t = jnp.tile(jnp.cos(emb), (B, 1))
    sin_t = jnp.tile(jnp.sin(emb), (B, 1))

    hid, xs, p_mat, offs, cnts = pl.pallas_call(
        _attn_router_body,
        out_shape=(
            jax.ShapeDtypeStruct((T, D), jnp.float32),
            jax.ShapeDtypeStruct((TP, D), jnp.float32),
            jax.ShapeDtypeStruct((T, TP), jnp.float32),
            jax.ShapeDtypeStruct((1, E), jnp.int32),
            jax.ShapeDtypeStruct((1, E), jnp.int32),
        ),
    )(h2, ln1_w.reshape(1, D), wq, wk, wv, q_norm_w.reshape(1, DH),
      k_norm_w.reshape(1, DH), wo, ln2_w.reshape(1, D), router_w,
      cos_t, sin_t)

    ys = pl.pallas_call(
        _moe_body,
        grid_spec=pltpu.PrefetchScalarGridSpec(
            num_scalar_prefetch=2,
            grid=(E,),
            in_specs=[
                pl.BlockSpec((TP, D), lambda e, offs, cnts: (0, 0)),
                pl.BlockSpec((1, D, 2 * F), lambda e, offs, cnts: (e, 0, 0)),
                pl.BlockSpec((1, F, D), lambda e, offs, cnts: (e, 0, 0)),
            ],
            out_specs=pl.BlockSpec((TP, D), lambda e, offs, cnts: (0, 0)),
        ),
        out_shape=jax.ShapeDtypeStruct((TP, D), jnp.float32),
    )(offs.reshape(E), cnts.reshape(E), xs, w_gate_up, w_down)

    out = pl.pallas_call(
        _unsort_body,
        out_shape=jax.ShapeDtypeStruct((T, D), jnp.float32),
    )(hid, p_mat, ys)
    return out.reshape(B, S, D)


# speedup vs baseline: 1.2512x; 1.0261x over previous
import functools
import math

import jax
import jax.numpy as jnp
from jax.experimental import pallas as pl
from jax.experimental.pallas import tpu as pltpu

D = 1024
H = 16
HK = 4
DH = 64
E = 16
F = 768
B = 32
S = 4
T = B * S
EPS = 1e-06
THETA = 1000000.0
TILE = 16
TP = 256


def _rms(x, w, eps=EPS):
    var = jnp.mean(x * x, axis=-1, keepdims=True)
    return x * jax.lax.rsqrt(var + eps) * w


def _attn_router_body(h_ref, ln1_ref, wq_ref, wk_ref, wv_ref, qn_ref, kn_ref,
                      wo_ref, ln2_ref, rw_ref, cos_ref, sin_ref,
                      hid_ref, xs_ref, p_ref, off_ref, cnt_ref):
    h = h_ref[...]
    x = _rms(h, ln1_ref[...])
    q2 = jnp.dot(x, wq_ref[...], preferred_element_type=jnp.float32)
    k2 = jnp.dot(x, wk_ref[...], preferred_element_type=jnp.float32)
    v2 = jnp.dot(x, wv_ref[...], preferred_element_type=jnp.float32)
    cos = cos_ref[...]
    sin = sin_ref[...]
    qn = qn_ref[...]
    kn = kn_ref[...]

    def rope(z):
        zr = jnp.concatenate([-z[:, DH // 2:], z[:, :DH // 2]], axis=1)
        return z * cos + zr * sin

    ks = []
    vs = []
    for j in range(HK):
        kj = rope(_rms(k2[:, j * DH:(j + 1) * DH], kn))
        ks.append(kj)
        vs.append(v2[:, j * DH:(j + 1) * DH])

    ti = jax.lax.broadcasted_iota(jnp.int32, (T, T), 0)
    tj = jax.lax.broadcasted_iota(jnp.int32, (T, T), 1)
    mask = (ti // S == tj // S) & (tj % S <= ti % S)
    neg = jnp.float32(-1e30)
    scale = jnp.float32(1.0 / math.sqrt(DH))

    outs = []
    for hh in range(H):
        j = hh // (H // HK)
        qh = rope(_rms(q2[:, hh * DH:(hh + 1) * DH], qn))
        sc = jax.lax.dot_general(qh, ks[j], (((1,), (1,)), ((), ())),
                                 preferred_element_type=jnp.float32) * scale
        sc = jnp.where(mask, sc, neg)
        sc = sc - jnp.max(sc, axis=1, keepdims=True)
        ex = jnp.exp(sc)
        at = ex / jnp.sum(ex, axis=1, keepdims=True)
        outs.append(jnp.dot(at, vs[j], preferred_element_type=jnp.float32))
    o2 = jnp.concatenate(outs, axis=1)
    hidden = h + jnp.dot(o2, wo_ref[...], preferred_element_type=jnp.float32)
    hid_ref[...] = hidden

    x2 = _rms(hidden, ln2_ref[...])
    logits = jnp.dot(x2, rw_ref[...], preferred_element_type=jnp.float32)

    eidx = jax.lax.broadcasted_iota(jnp.int32, (T, E), 1)
    rowmax = jnp.max(logits, axis=1, keepdims=True)
    assign = jnp.min(jnp.where(logits == rowmax, eidx, E), axis=1,
                     keepdims=True)
    onehot = (eidx == assign).astype(jnp.float32)
    counts = jnp.sum(onehot, axis=0, keepdims=True)
    padded = jnp.float32(8.0) * jnp.ceil(counts * jnp.float32(1.0 / 8.0))
    er = jax.lax.broadcasted_iota(jnp.int32, (E, E), 0)
    ec = jax.lax.broadcasted_iota(jnp.int32, (E, E), 1)
    lt_e = (er < ec).astype(jnp.float32)
    offsets = jnp.dot(padded, lt_e, preferred_element_type=jnp.float32)
    lt_t = (tj < ti).astype(jnp.float32)
    intra = jnp.dot(lt_t, onehot, preferred_element_type=jnp.float32)
    rank = jnp.sum(onehot * (offsets + intra), axis=1, keepdims=True)
    rj = jax.lax.broadcasted_iota(jnp.int32, (T, TP), 1)
    p_mat = (rank.astype(jnp.int32) == rj).astype(jnp.float32)
    p_ref[...] = p_mat
    xs_ref[...] = jnp.dot(p_mat.T, x2, preferred_element_type=jnp.float32)
    off_ref[...] = (offsets * jnp.float32(1.0 / 8.0)).astype(jnp.int32)
    cnt_ref[...] = counts.astype(jnp.int32)


NBUF = 3


def _moe_body(off8_sm, cnt_sm, xs_ref, wgu_hbm, wd_hbm, p_ref, hid_ref,
              out_ref, gu_buf, d_buf, y_ref, gu_sem, d_sem):
    e = pl.program_id(0)
    slot = jax.lax.rem(e, NBUF)

    def start_copy(idx, s):
        pltpu.make_async_copy(wgu_hbm.at[idx], gu_buf.at[s],
                              gu_sem.at[s]).start()
        pltpu.make_async_copy(wd_hbm.at[idx], d_buf.at[s], d_sem.at[s]).start()

    @pl.when(e == 0)
    def _init():
        y_ref[...] = jnp.zeros_like(y_ref)
        for i in range(NBUF):
            start_copy(i, i)

    pltpu.make_async_copy(wgu_hbm.at[e], gu_buf.at[slot],
                          gu_sem.at[slot]).wait()
    pltpu.make_async_copy(wd_hbm.at[e], d_buf.at[slot], d_sem.at[slot]).wait()

    off8 = off8_sm[e]
    cnt = cnt_sm[e]
    ntiles = (cnt + TILE - 1) // TILE
    wgu = gu_buf[slot]
    wd = d_buf[slot]
    rid = jax.lax.broadcasted_iota(jnp.int32, (TILE, 1), 0)

    def tile_step(t, _):
        base = off8 * 8 + t * TILE
        xb = xs_ref[pl.ds(base, TILE), :]
        gu = jnp.dot(xb, wgu, preferred_element_type=jnp.float32)
        gate = gu[:, :F]
        up = gu[:, F:]
        act = gate * jax.lax.logistic(gate) * up
        y = jnp.dot(act, wd, preferred_element_type=jnp.float32)
        valid = t * TILE + rid < cnt
        cur = y_ref[pl.ds(base, TILE), :]
        y_ref[pl.ds(base, TILE), :] = jnp.where(valid, y, cur)
        return 0

    jax.lax.fori_loop(0, ntiles, tile_step, 0)

    @pl.when(e + NBUF < E)
    def _next():
        start_copy(e + NBUF, slot)

    @pl.when(e == E - 1)
    def _finish():
        out_ref[...] = hid_ref[...] + jnp.dot(
            p_ref[...], y_ref[...], preferred_element_type=jnp.float32)


@jax.jit
def kernel(hidden_states, ln1_w, wq, wk, wv, q_norm_w, k_norm_w, wo, ln2_w,
           router_w, w_gate_up, w_down):
    h2 = hidden_states.reshape(T, D)
    pos = jnp.arange(S, dtype=jnp.float32)
    inv = 1.0 / (THETA ** (jnp.arange(0, DH, 2, dtype=jnp.float32) / DH))
    freqs = pos[:, None] * inv[None, :]
    emb = jnp.concatenate([freqs, freqs], axis=-1)
    cos_t = jnp.tile(jnp.cos(emb), (B, 1))
    sin_t = jnp.tile(jnp.sin(emb), (B, 1))

    hid, xs, p_mat, offs, cnts = pl.pallas_call(
        _attn_router_body,
        out_shape=(
            jax.ShapeDtypeStruct((T, D), jnp.float32),
            jax.ShapeDtypeStruct((TP, D), jnp.float32),
            jax.ShapeDtypeStruct((T, TP), jnp.float32),
            jax.ShapeDtypeStruct((1, E), jnp.int32),
            jax.ShapeDtypeStruct((1, E), jnp.int32),
        ),
    )(h2, ln1_w.reshape(1, D), wq, wk, wv, q_norm_w.reshape(1, DH),
      k_norm_w.reshape(1, DH), wo, ln2_w.reshape(1, D), router_w,
      cos_t, sin_t)

    out = pl.pallas_call(
        _moe_body,
        grid_spec=pltpu.PrefetchScalarGridSpec(
            num_scalar_prefetch=2,
            grid=(E,),
            in_specs=[
                pl.BlockSpec((TP, D), lambda e, offs, cnts: (0, 0)),
                pl.BlockSpec(memory_space=pltpu.MemorySpace.HBM),
                pl.BlockSpec(memory_space=pltpu.MemorySpace.HBM),
                pl.BlockSpec((T, TP), lambda e, offs, cnts: (0, 0)),
                pl.BlockSpec((T, D), lambda e, offs, cnts: (0, 0)),
            ],
            out_specs=pl.BlockSpec((T, D), lambda e, offs, cnts: (0, 0)),
            scratch_shapes=[
                pltpu.VMEM((NBUF, D, 2 * F), jnp.float32),
                pltpu.VMEM((NBUF, F, D), jnp.float32),
                pltpu.VMEM((TP, D), jnp.float32),
                pltpu.SemaphoreType.DMA((NBUF,)),
                pltpu.SemaphoreType.DMA((NBUF,)),
            ],
        ),
        out_shape=jax.ShapeDtypeStruct((T, D), jnp.float32),
    )(offs.reshape(E), cnts.reshape(E), xs, w_gate_up, w_down, p_mat, hid)
    return out.reshape(B, S, D)


# speedup vs baseline: 1.3496x; 1.0786x over previous
import functools
import math

import jax
import jax.numpy as jnp
from jax.experimental import pallas as pl
from jax.experimental.pallas import tpu as pltpu

D = 1024
H = 16
HK = 4
DH = 64
E = 16
F = 768
B = 32
S = 4
T = B * S
EPS = 1e-06
THETA = 1000000.0
TILE = 16
TP = 256


def _rms(x, w, eps=EPS):
    var = jnp.mean(x * x, axis=-1, keepdims=True)
    return x * jax.lax.rsqrt(var + eps) * w


def _attn_router_body(h_ref, ln1_ref, wq_ref, wk_ref, wv_ref, qn_ref, kn_ref,
                      wo_ref, ln2_ref, rw_ref,
                      hid_ref, xs_ref, p_ref, off_ref, cnt_ref):
    h = h_ref[...].reshape(T, D)
    x = _rms(h, ln1_ref[...])
    q2 = jnp.dot(x, wq_ref[...], preferred_element_type=jnp.float32)
    k2 = jnp.dot(x, wk_ref[...], preferred_element_type=jnp.float32)
    v2 = jnp.dot(x, wv_ref[...], preferred_element_type=jnp.float32)
    rowpos = jax.lax.broadcasted_iota(jnp.int32, (T, DH), 0) % S
    colf = jax.lax.broadcasted_iota(jnp.int32, (T, DH), 1) % (DH // 2)
    inv = jnp.exp(colf.astype(jnp.float32) *
                  jnp.float32(-2.0 * math.log(THETA) / DH))
    ang = rowpos.astype(jnp.float32) * inv
    cos = jnp.cos(ang)
    sin = jnp.sin(ang)
    qn = qn_ref[...]
    kn = kn_ref[...]

    def rope(z):
        zr = jnp.concatenate([-z[:, DH // 2:], z[:, :DH // 2]], axis=1)
        return z * cos + zr * sin

    ks = []
    vs = []
    for j in range(HK):
        kj = rope(_rms(k2[:, j * DH:(j + 1) * DH], kn))
        ks.append(kj)
        vs.append(v2[:, j * DH:(j + 1) * DH])

    ti = jax.lax.broadcasted_iota(jnp.int32, (T, T), 0)
    tj = jax.lax.broadcasted_iota(jnp.int32, (T, T), 1)
    mask = (ti // S == tj // S) & (tj % S <= ti % S)
    neg = jnp.float32(-1e30)
    scale = jnp.float32(1.0 / math.sqrt(DH))

    outs = []
    for hh in range(H):
        j = hh // (H // HK)
        qh = rope(_rms(q2[:, hh * DH:(hh + 1) * DH], qn))
        sc = jax.lax.dot_general(qh, ks[j], (((1,), (1,)), ((), ())),
                                 preferred_element_type=jnp.float32) * scale
        sc = jnp.where(mask, sc, neg)
        sc = sc - jnp.max(sc, axis=1, keepdims=True)
        ex = jnp.exp(sc)
        at = ex / jnp.sum(ex, axis=1, keepdims=True)
        outs.append(jnp.dot(at, vs[j], preferred_element_type=jnp.float32))
    o2 = jnp.concatenate(outs, axis=1)
    hidden = h + jnp.dot(o2, wo_ref[...], preferred_element_type=jnp.float32)
    hid_ref[...] = hidden

    x2 = _rms(hidden, ln2_ref[...])
    logits = jnp.dot(x2, rw_ref[...], preferred_element_type=jnp.float32)

    eidx = jax.lax.broadcasted_iota(jnp.int32, (T, E), 1)
    rowmax = jnp.max(logits, axis=1, keepdims=True)
    assign = jnp.min(jnp.where(logits == rowmax, eidx, E), axis=1,
                     keepdims=True)
    onehot = (eidx == assign).astype(jnp.float32)
    counts = jnp.sum(onehot, axis=0, keepdims=True)
    padded = jnp.float32(8.0) * jnp.ceil(counts * jnp.float32(1.0 / 8.0))
    er = jax.lax.broadcasted_iota(jnp.int32, (E, E), 0)
    ec = jax.lax.broadcasted_iota(jnp.int32, (E, E), 1)
    lt_e = (er < ec).astype(jnp.float32)
    offsets = jnp.dot(padded, lt_e, preferred_element_type=jnp.float32)
    lt_t = (tj < ti).astype(jnp.float32)
    intra = jnp.dot(lt_t, onehot, preferred_element_type=jnp.float32)
    rank = jnp.sum(onehot * (offsets + intra), axis=1, keepdims=True)
    rj = jax.lax.broadcasted_iota(jnp.int32, (T, TP), 1)
    p_mat = (rank.astype(jnp.int32) == rj).astype(jnp.float32)
    p_ref[...] = p_mat
    xs_ref[...] = jnp.dot(p_mat.T, x2, preferred_element_type=jnp.float32)
    off_ref[...] = (offsets * jnp.float32(1.0 / 8.0)).astype(jnp.int32)
    cnt_ref[...] = counts.astype(jnp.int32)


NBUF = 3


def _moe_body(off8_sm, cnt_sm, xs_ref, wgu_hbm, wd_hbm, p_ref, hid_ref,
              out_ref, gu_buf, d_buf, y_ref, gu_sem, d_sem):
    e = pl.program_id(0)
    slot = jax.lax.rem(e, NBUF)
    off8 = off8_sm[0, e]
    cnt = cnt_sm[0, e]

    def start_copy(idx, s):
        pltpu.make_async_copy(wgu_hbm.at[idx], gu_buf.at[s],
                              gu_sem.at[s]).start()
        pltpu.make_async_copy(wd_hbm.at[idx], d_buf.at[s], d_sem.at[s]).start()

    @pl.when(e == 0)
    def _init():
        y_ref[...] = jnp.zeros_like(y_ref)
        for i in range(NBUF):
            start_copy(i, i)

    pltpu.make_async_copy(wgu_hbm.at[e], gu_buf.at[slot],
                          gu_sem.at[slot]).wait()
    pltpu.make_async_copy(wd_hbm.at[e], d_buf.at[slot], d_sem.at[slot]).wait()

    ntiles = (cnt + TILE - 1) // TILE
    wgu = gu_buf[slot]
    wd = d_buf[slot]
    rid = jax.lax.broadcasted_iota(jnp.int32, (TILE, 1), 0)

    def tile_step(t, _):
        base = off8 * 8 + t * TILE
        xb = xs_ref[pl.ds(base, TILE), :]
        gu = jnp.dot(xb, wgu, preferred_element_type=jnp.float32)
        gate = gu[:, :F]
        up = gu[:, F:]
        act = gate * jax.lax.logistic(gate) * up
        y = jnp.dot(act, wd, preferred_element_type=jnp.float32)
        valid = t * TILE + rid < cnt
        cur = y_ref[pl.ds(base, TILE), :]
        y_ref[pl.ds(base, TILE), :] = jnp.where(valid, y, cur)
        return 0

    jax.lax.fori_loop(0, ntiles, tile_step, 0)

    @pl.when(e + NBUF < E)
    def _next():
        start_copy(e + NBUF, slot)

    @pl.when(e == E - 1)
    def _finish():
        res = hid_ref[...] + jnp.dot(
            p_ref[...], y_ref[...], preferred_element_type=jnp.float32)
        out_ref[...] = res.reshape(B, S, D)


@jax.jit
def kernel(hidden_states, ln1_w, wq, wk, wv, q_norm_w, k_norm_w, wo, ln2_w,
           router_w, w_gate_up, w_down):
    hid, xs, p_mat, offs, cnts = pl.pallas_call(
        _attn_router_body,
        out_shape=(
            jax.ShapeDtypeStruct((T, D), jnp.float32),
            jax.ShapeDtypeStruct((TP, D), jnp.float32),
            jax.ShapeDtypeStruct((T, TP), jnp.float32),
            jax.ShapeDtypeStruct((1, E), jnp.int32),
            jax.ShapeDtypeStruct((1, E), jnp.int32),
        ),
    )(hidden_states, ln1_w.reshape(1, D), wq, wk, wv, q_norm_w.reshape(1, DH),
      k_norm_w.reshape(1, DH), wo, ln2_w.reshape(1, D), router_w)

    out = pl.pallas_call(
        _moe_body,
        grid_spec=pltpu.PrefetchScalarGridSpec(
            num_scalar_prefetch=2,
            grid=(E,),
            in_specs=[
                pl.BlockSpec((TP, D), lambda e, offs, cnts: (0, 0)),
                pl.BlockSpec(memory_space=pltpu.MemorySpace.HBM),
                pl.BlockSpec(memory_space=pltpu.MemorySpace.HBM),
                pl.BlockSpec((T, TP), lambda e, offs, cnts: (0, 0)),
                pl.BlockSpec((T, D), lambda e, offs, cnts: (0, 0)),
            ],
            out_specs=pl.BlockSpec((B, S, D), lambda e, offs, cnts: (0, 0, 0)),
            scratch_shapes=[
                pltpu.VMEM((NBUF, D, 2 * F), jnp.float32),
                pltpu.VMEM((NBUF, F, D), jnp.float32),
                pltpu.VMEM((TP, D), jnp.float32),
                pltpu.SemaphoreType.DMA((NBUF,)),
                pltpu.SemaphoreType.DMA((NBUF,)),
            ],
        ),
        out_shape=jax.ShapeDtypeStruct((B, S, D), jnp.float32),
    )(offs, cnts, xs, w_gate_up, w_down, p_mat, hid)
    return out


# speedup vs baseline: 1.3520x; 1.0018x over previous
import functools
import math

import jax
import jax.numpy as jnp
from jax.experimental import pallas as pl
from jax.experimental.pallas import tpu as pltpu

D = 1024
H = 16
HK = 4
DH = 64
E = 16
F = 768
B = 32
S = 4
T = B * S
EPS = 1e-06
THETA = 1000000.0
TILE = 16
TP = 256


def _rms(x, w, eps=EPS):
    var = jnp.mean(x * x, axis=-1, keepdims=True)
    return x * jax.lax.rsqrt(var + eps) * w


def _attn_router_body(h_ref, ln1_ref, wq_ref, wk_ref, wv_ref, qn_ref, kn_ref,
                      wo_ref, ln2_ref, rw_ref,
                      hid_ref, xs_ref, p_ref, off_ref, cnt_ref):
    h = h_ref[...].reshape(T, D)
    x = _rms(h, ln1_ref[...])
    q2 = jnp.dot(x, wq_ref[...], preferred_element_type=jnp.float32)
    k2 = jnp.dot(x, wk_ref[...], preferred_element_type=jnp.float32)
    v2 = jnp.dot(x, wv_ref[...], preferred_element_type=jnp.float32)
    rowpos = jax.lax.broadcasted_iota(jnp.int32, (T, DH), 0) % S
    colf = jax.lax.broadcasted_iota(jnp.int32, (T, DH), 1) % (DH // 2)
    inv = jnp.exp(colf.astype(jnp.float32) *
                  jnp.float32(-2.0 * math.log(THETA) / DH))
    ang = rowpos.astype(jnp.float32) * inv
    cos = jnp.cos(ang)
    sin = jnp.sin(ang)
    qn = qn_ref[...]
    kn = kn_ref[...]

    def rope(z):
        zr = jnp.concatenate([-z[:, DH // 2:], z[:, :DH // 2]], axis=1)
        return z * cos + zr * sin

    ks = []
    vs = []
    for j in range(HK):
        kj = rope(_rms(k2[:, j * DH:(j + 1) * DH], kn))
        ks.append(kj)
        vs.append(v2[:, j * DH:(j + 1) * DH])

    ti = jax.lax.broadcasted_iota(jnp.int32, (T, T), 0)
    tj = jax.lax.broadcasted_iota(jnp.int32, (T, T), 1)
    mask = (ti // S == tj // S) & (tj % S <= ti % S)
    neg = jnp.float32(-1e30)
    scale = jnp.float32(1.0 / math.sqrt(DH))

    outs = []
    for hh in range(H):
        j = hh // (H // HK)
        qh = rope(_rms(q2[:, hh * DH:(hh + 1) * DH], qn))
        sc = jax.lax.dot_general(qh, ks[j], (((1,), (1,)), ((), ())),
                                 preferred_element_type=jnp.float32) * scale
        sc = jnp.where(mask, sc, neg)
        sc = sc - jnp.max(sc, axis=1, keepdims=True)
        ex = jnp.exp(sc)
        at = ex / jnp.sum(ex, axis=1, keepdims=True)
        outs.append(jnp.dot(at, vs[j], preferred_element_type=jnp.float32))
    o2 = jnp.concatenate(outs, axis=1)
    hidden = h + jnp.dot(o2, wo_ref[...], preferred_element_type=jnp.float32)
    hid_ref[...] = hidden

    x2 = _rms(hidden, ln2_ref[...])
    logits = jnp.dot(x2, rw_ref[...], preferred_element_type=jnp.float32)

    eidx = jax.lax.broadcasted_iota(jnp.int32, (T, E), 1)
    rowmax = jnp.max(logits, axis=1, keepdims=True)
    assign = jnp.min(jnp.where(logits == rowmax, eidx, E), axis=1,
                     keepdims=True)
    onehot = (eidx == assign).astype(jnp.float32)
    counts = jnp.sum(onehot, axis=0, keepdims=True)
    padded = jnp.float32(8.0) * jnp.ceil(counts * jnp.float32(1.0 / 8.0))
    er = jax.lax.broadcasted_iota(jnp.int32, (E, E), 0)
    ec = jax.lax.broadcasted_iota(jnp.int32, (E, E), 1)
    lt_e = (er < ec).astype(jnp.float32)
    offsets = jnp.dot(padded, lt_e, preferred_element_type=jnp.float32)
    lt_t = (tj < ti).astype(jnp.float32)
    intra = jnp.dot(lt_t, onehot, preferred_element_type=jnp.float32)
    rank = jnp.sum(onehot * (offsets + intra), axis=1, keepdims=True)
    rj = jax.lax.broadcasted_iota(jnp.int32, (T, TP), 1)
    p_mat = (rank.astype(jnp.int32) == rj).astype(jnp.float32)
    p_ref[...] = p_mat
    xs_ref[...] = jnp.dot(p_mat.T, x2, preferred_element_type=jnp.float32)
    off_ref[...] = (offsets * jnp.float32(1.0 / 8.0)).astype(jnp.int32)
    cnt_ref[...] = counts.astype(jnp.int32)


NBUF = 3


def _moe_body(off8_sm, cnt_sm, xs_ref, wgu_hbm, wd_hbm, p_ref, hid_ref,
              out_ref, gu_buf, d_buf, y_ref, gu_sem, d_sem):
    e = pl.program_id(0)
    slot = jax.lax.rem(e, NBUF)
    off8 = off8_sm[0, e]
    cnt = cnt_sm[0, e]

    def _gu_copy(idx, s, half):
        rows = pl.ds(half * (D // 2), D // 2)
        return pltpu.make_async_copy(wgu_hbm.at[idx, rows, :],
                                     gu_buf.at[s, rows, :],
                                     gu_sem.at[s, half])

    def _d_copy(idx, s, half):
        rows = pl.ds(half * (F // 2), F // 2)
        return pltpu.make_async_copy(wd_hbm.at[idx, rows, :],
                                     d_buf.at[s, rows, :],
                                     d_sem.at[s, half])

    def start_copy(idx, s):
        for half in range(2):
            _gu_copy(idx, s, half).start()
            _d_copy(idx, s, half).start()

    @pl.when(e == 0)
    def _init():
        y_ref[...] = jnp.zeros_like(y_ref)
        for i in range(NBUF):
            start_copy(i, i)

    for half in range(2):
        _gu_copy(e, slot, half).wait()
        _d_copy(e, slot, half).wait()

    ntiles = (cnt + TILE - 1) // TILE
    wgu = gu_buf[slot]
    wd = d_buf[slot]
    rid = jax.lax.broadcasted_iota(jnp.int32, (TILE, 1), 0)

    def tile_step(t, _):
        base = off8 * 8 + t * TILE
        xb = xs_ref[pl.ds(base, TILE), :]
        gu = jnp.dot(xb, wgu, preferred_element_type=jnp.float32)
        gate = gu[:, :F]
        up = gu[:, F:]
        act = gate * jax.lax.logistic(gate) * up
        y = jnp.dot(act, wd, preferred_element_type=jnp.float32)
        valid = t * TILE + rid < cnt
        cur = y_ref[pl.ds(base, TILE), :]
        y_ref[pl.ds(base, TILE), :] = jnp.where(valid, y, cur)
        return 0

    jax.lax.fori_loop(0, ntiles, tile_step, 0)

    @pl.when(e + NBUF < E)
    def _next():
        start_copy(e + NBUF, slot)

    @pl.when(e == E - 1)
    def _finish():
        res = hid_ref[...] + jnp.dot(
            p_ref[...], y_ref[...], preferred_element_type=jnp.float32)
        out_ref[...] = res.reshape(B, S, D)


@jax.jit
def kernel(hidden_states, ln1_w, wq, wk, wv, q_norm_w, k_norm_w, wo, ln2_w,
           router_w, w_gate_up, w_down):
    hid, xs, p_mat, offs, cnts = pl.pallas_call(
        _attn_router_body,
        out_shape=(
            jax.ShapeDtypeStruct((T, D), jnp.float32),
            jax.ShapeDtypeStruct((TP, D), jnp.float32),
            jax.ShapeDtypeStruct((T, TP), jnp.float32),
            jax.ShapeDtypeStruct((1, E), jnp.int32),
            jax.ShapeDtypeStruct((1, E), jnp.int32),
        ),
    )(hidden_states, ln1_w.reshape(1, D), wq, wk, wv, q_norm_w.reshape(1, DH),
      k_norm_w.reshape(1, DH), wo, ln2_w.reshape(1, D), router_w)

    out = pl.pallas_call(
        _moe_body,
        grid_spec=pltpu.PrefetchScalarGridSpec(
            num_scalar_prefetch=2,
            grid=(E,),
            in_specs=[
                pl.BlockSpec((TP, D), lambda e, offs, cnts: (0, 0)),
                pl.BlockSpec(memory_space=pltpu.MemorySpace.HBM),
                pl.BlockSpec(memory_space=pltpu.MemorySpace.HBM),
                pl.BlockSpec((T, TP), lambda e, offs, cnts: (0, 0)),
                pl.BlockSpec((T, D), lambda e, offs, cnts: (0, 0)),
            ],
            out_specs=pl.BlockSpec((B, S, D), lambda e, offs, cnts: (0, 0, 0)),
            scratch_shapes=[
                pltpu.VMEM((NBUF, D, 2 * F), jnp.float32),
                pltpu.VMEM((NBUF, F, D), jnp.float32),
                pltpu.VMEM((TP, D), jnp.float32),
                pltpu.SemaphoreType.DMA((NBUF, 2)),
                pltpu.SemaphoreType.DMA((NBUF, 2)),
            ],
        ),
        out_shape=jax.ShapeDtypeStruct((B, S, D), jnp.float32),
    )(offs, cnts, xs, w_gate_up, w_down, p_mat, hid)
    return out


# speedup vs baseline: 1.5956x; 1.1802x over previous
import functools
import math

import jax
import jax.numpy as jnp
from jax.experimental import pallas as pl
from jax.experimental.pallas import tpu as pltpu

D = 1024
H = 16
HK = 4
DH = 64
E = 16
F = 768
B = 32
S = 4
T = B * S
EPS = 1e-06
THETA = 1000000.0
TILE = 16
TP = 256
NBUF = 3


def _rms(x, w, eps=EPS):
    var = jnp.mean(x * x, axis=-1, keepdims=True)
    return x * jax.lax.rsqrt(var + eps) * w


def _fused_body(h_ref, ln1_ref, wq_ref, wk_ref, wv_ref, qn_ref, kn_ref,
                wo_ref, ln2_ref, rw_ref, wgu_hbm, wd_hbm, out_ref,
                x2_scr, hid_scr, p_scr, off_scr, cnt_scr, y_scr,
                gu_buf, d_buf, gu_sem, d_sem):
    i = pl.program_id(0)

    def _gu_copy(idx, s, half):
        rows = pl.ds(half * (D // 2), D // 2)
        return pltpu.make_async_copy(wgu_hbm.at[idx, rows, :],
                                     gu_buf.at[s, rows, :],
                                     gu_sem.at[s, half])

    def _d_copy(idx, s, half):
        rows = pl.ds(half * (F // 2), F // 2)
        return pltpu.make_async_copy(wd_hbm.at[idx, rows, :],
                                     d_buf.at[s, rows, :],
                                     d_sem.at[s, half])

    def start_copy(idx, s):
        for half in range(2):
            _gu_copy(idx, s, half).start()
            _d_copy(idx, s, half).start()

    @pl.when(i == 0)
    def _attn_router():
        for b in range(NBUF):
            start_copy(b, b)
        y_scr[...] = jnp.zeros_like(y_scr)

        h = h_ref[...].reshape(T, D)
        x = _rms(h, ln1_ref[...])
        q2 = jnp.dot(x, wq_ref[...], preferred_element_type=jnp.float32)
        k2 = jnp.dot(x, wk_ref[...], preferred_element_type=jnp.float32)
        v2 = jnp.dot(x, wv_ref[...], preferred_element_type=jnp.float32)
        rowpos = jax.lax.broadcasted_iota(jnp.int32, (T, DH), 0) % S
        colf = jax.lax.broadcasted_iota(jnp.int32, (T, DH), 1) % (DH // 2)
        inv = jnp.exp(colf.astype(jnp.float32) *
                      jnp.float32(-2.0 * math.log(THETA) / DH))
        ang = rowpos.astype(jnp.float32) * inv
        cos = jnp.cos(ang)
        sin = jnp.sin(ang)
        qn = qn_ref[...]
        kn = kn_ref[...]

        def rope(z):
            zr = jnp.concatenate([-z[:, DH // 2:], z[:, :DH // 2]], axis=1)
            return z * cos + zr * sin

        ks = []
        vs = []
        for j in range(HK):
            kj = rope(_rms(k2[:, j * DH:(j + 1) * DH], kn))
            ks.append(kj)
            vs.append(v2[:, j * DH:(j + 1) * DH])

        ti = jax.lax.broadcasted_iota(jnp.int32, (T, T), 0)
        tj = jax.lax.broadcasted_iota(jnp.int32, (T, T), 1)
        mask = (ti // S == tj // S) & (tj % S <= ti % S)
        neg = jnp.float32(-1e30)
        scale = jnp.float32(1.0 / math.sqrt(DH))

        outs = []
        for hh in range(H):
            j = hh // (H // HK)
            qh = rope(_rms(q2[:, hh * DH:(hh + 1) * DH], qn))
            sc = jax.lax.dot_general(qh, ks[j], (((1,), (1,)), ((), ())),
                                     preferred_element_type=jnp.float32)
            sc = jnp.where(mask, sc * scale, neg)
            ex = jnp.exp(sc - jnp.max(sc, axis=1, keepdims=True))
            at = ex / jnp.sum(ex, axis=1, keepdims=True)
            outs.append(jnp.dot(at, vs[j], preferred_element_type=jnp.float32))
        o2 = jnp.concatenate(outs, axis=1)
        hidden = h + jnp.dot(o2, wo_ref[...],
                             preferred_element_type=jnp.float32)
        hid_scr[...] = hidden

        x2 = _rms(hidden, ln2_ref[...])
        logits = jnp.dot(x2, rw_ref[...], preferred_element_type=jnp.float32)

        eidx = jax.lax.broadcasted_iota(jnp.int32, (T, E), 1)
        rowmax = jnp.max(logits, axis=1, keepdims=True)
        assign = jnp.min(jnp.where(logits == rowmax, eidx, E), axis=1,
                         keepdims=True)
        onehot = (eidx == assign).astype(jnp.float32)
        counts = jnp.sum(onehot, axis=0, keepdims=True)
        padded = jnp.float32(8.0) * jnp.ceil(counts * jnp.float32(1.0 / 8.0))
        er = jax.lax.broadcasted_iota(jnp.int32, (E, E), 0)
        ec = jax.lax.broadcasted_iota(jnp.int32, (E, E), 1)
        lt_e = (er < ec).astype(jnp.float32)
        offsets = jnp.dot(padded, lt_e,
                          preferred_element_type=jnp.float32)
        lt_t = (tj < ti).astype(jnp.float32)
        intra = jnp.dot(lt_t, onehot,
                        preferred_element_type=jnp.float32)
        rank = jnp.sum(onehot * (offsets + intra), axis=1,
                       keepdims=True)
        rj = jax.lax.broadcasted_iota(jnp.int32, (T, TP), 1)
        p_mat = (rank.astype(jnp.int32) == rj).astype(jnp.float32)
        p_scr[...] = p_mat
        x2_scr[...] = jnp.dot(p_mat.T, x2, preferred_element_type=jnp.float32)
        se = jax.lax.broadcasted_iota(jnp.int32, (E, E * 128), 0)
        sl = jax.lax.broadcasted_iota(jnp.int32, (E, E * 128), 1)
        spread = (sl // 128 == se).astype(jnp.float32)
        off_scr[...] = jnp.dot(offsets * jnp.float32(1.0 / 8.0), spread,
                               preferred_element_type=jnp.float32
                               ).astype(jnp.int32)
        cnt_scr[...] = jnp.dot(counts, spread,
                               preferred_element_type=jnp.float32
                               ).astype(jnp.int32)

    @pl.when(i > 0)
    def _expert():
        e = i - 1
        slot = jax.lax.rem(e, NBUF)
        off8 = off_scr[0, e * 128]
        cnt = cnt_scr[0, e * 128]
        for half in range(2):
            _gu_copy(e, slot, half).wait()
            _d_copy(e, slot, half).wait()

        ntiles = (cnt + TILE - 1) // TILE
        wgu = gu_buf[slot]
        wd = d_buf[slot]
        rid = jax.lax.broadcasted_iota(jnp.int32, (TILE, 1), 0)

        def tile_step(t, _):
            base = off8 * 8 + t * TILE
            xb = x2_scr[pl.ds(base, TILE), :]
            gu = jnp.dot(xb, wgu, preferred_element_type=jnp.float32)
            gate = gu[:, :F]
            up = gu[:, F:]
            act = gate * jax.lax.logistic(gate) * up
            y = jnp.dot(act, wd, preferred_element_type=jnp.float32)
            valid = t * TILE + rid < cnt
            cur = y_scr[pl.ds(base, TILE), :]
            y_scr[pl.ds(base, TILE), :] = jnp.where(valid, y, cur)
            return 0

        jax.lax.fori_loop(0, ntiles, tile_step, 0)

        @pl.when(e + NBUF < E)
        def _next():
            start_copy(e + NBUF, slot)

    @pl.when(i == E)
    def _finish():
        res = hid_scr[...] + jnp.dot(p_scr[...], y_scr[...],
                                     preferred_element_type=jnp.float32)
        out_ref[...] = res.reshape(B, S, D)


@jax.jit
def kernel(hidden_states, ln1_w, wq, wk, wv, q_norm_w, k_norm_w, wo, ln2_w,
           router_w, w_gate_up, w_down):
    whole = lambda e: (0,) * 2
    out = pl.pallas_call(
        _fused_body,
        grid=(E + 1,),
        in_specs=[
            pl.BlockSpec((B, S, D), lambda e: (0, 0, 0)),
            pl.BlockSpec((1, D), whole),
            pl.BlockSpec((D, H * DH), whole),
            pl.BlockSpec((D, HK * DH), whole),
            pl.BlockSpec((D, HK * DH), whole),
            pl.BlockSpec((1, DH), whole),
            pl.BlockSpec((1, DH), whole),
            pl.BlockSpec((H * DH, D), whole),
            pl.BlockSpec((1, D), whole),
            pl.BlockSpec((D, E), whole),
            pl.BlockSpec(memory_space=pltpu.MemorySpace.HBM),
            pl.BlockSpec(memory_space=pltpu.MemorySpace.HBM),
        ],
        out_specs=pl.BlockSpec((B, S, D), lambda e: (0, 0, 0)),
        scratch_shapes=[
            pltpu.VMEM((TP, D), jnp.float32),
            pltpu.VMEM((T, D), jnp.float32),
            pltpu.VMEM((T, TP), jnp.float32),
            pltpu.VMEM((1, E * 128), jnp.int32),
            pltpu.VMEM((1, E * 128), jnp.int32),
            pltpu.VMEM((TP, D), jnp.float32),
            pltpu.VMEM((NBUF, D, 2 * F), jnp.float32),
            pltpu.VMEM((NBUF, F, D), jnp.float32),
            pltpu.SemaphoreType.DMA((NBUF, 2)),
            pltpu.SemaphoreType.DMA((NBUF, 2)),
        ],
        out_shape=jax.ShapeDtypeStruct((B, S, D), jnp.float32),
    )(hidden_states, ln1_w.reshape(1, D), wq, wk, wv,
      q_norm_w.reshape(1, DH), k_norm_w.reshape(1, DH), wo,
      ln2_w.reshape(1, D), router_w, w_gate_up, w_down)
    return out


# speedup vs baseline: 1.6503x; 1.0343x over previous
import functools
import math

import jax
import jax.numpy as jnp
from jax.experimental import pallas as pl
from jax.experimental.pallas import tpu as pltpu

D = 1024
H = 16
HK = 4
DH = 64
E = 16
F = 768
B = 32
S = 4
T = B * S
EPS = 1e-06
THETA = 1000000.0
TILE = 16
TP = 256
NBUF = 4


def _rms(x, w, eps=EPS):
    var = jnp.mean(x * x, axis=-1, keepdims=True)
    return x * jax.lax.rsqrt(var + eps) * w


def _fused_body(h_ref, ln1_ref, wq_ref, wk_ref, wv_ref, qn_ref, kn_ref,
                wo_ref, ln2_ref, rw_ref, wgu_hbm, wd_hbm, out_ref,
                x2_scr, p_scr, off_scr, cnt_scr, y_scr,
                gu_buf, d_buf, gu_sem, d_sem):
    i = pl.program_id(0)

    def _gu_copy(idx, s, half):
        rows = pl.ds(half * (D // 2), D // 2)
        return pltpu.make_async_copy(wgu_hbm.at[idx, rows, :],
                                     gu_buf.at[s, rows, :],
                                     gu_sem.at[s, half])

    def _d_copy(idx, s, half):
        rows = pl.ds(half * (F // 2), F // 2)
        return pltpu.make_async_copy(wd_hbm.at[idx, rows, :],
                                     d_buf.at[s, rows, :],
                                     d_sem.at[s, half])

    def start_copy(idx, s):
        for half in range(2):
            _gu_copy(idx, s, half).start()
            _d_copy(idx, s, half).start()

    @pl.when(i == 0)
    def _attn_router():
        for b in range(NBUF):
            start_copy(b, b)
        y_scr[...] = jnp.zeros_like(y_scr)

        h = h_ref[...].reshape(T, D)
        x = _rms(h, ln1_ref[...])
        q2 = jnp.dot(x, wq_ref[...], preferred_element_type=jnp.float32)
        k2 = jnp.dot(x, wk_ref[...], preferred_element_type=jnp.float32)
        v2 = jnp.dot(x, wv_ref[...], preferred_element_type=jnp.float32)
        rowpos = jax.lax.broadcasted_iota(jnp.int32, (T, DH), 0) % S
        colf = jax.lax.broadcasted_iota(jnp.int32, (T, DH), 1) % (DH // 2)
        inv = jnp.exp(colf.astype(jnp.float32) *
                      jnp.float32(-2.0 * math.log(THETA) / DH))
        ang = rowpos.astype(jnp.float32) * inv
        cos = jnp.cos(ang)
        sin = jnp.sin(ang)
        qn = qn_ref[...]
        kn = kn_ref[...]

        def rope(z):
            zr = jnp.concatenate([-z[:, DH // 2:], z[:, :DH // 2]], axis=1)
            return z * cos + zr * sin

        ks = []
        vs = []
        for j in range(HK):
            kj = rope(_rms(k2[:, j * DH:(j + 1) * DH], kn))
            ks.append(kj)
            vs.append(v2[:, j * DH:(j + 1) * DH])

        ti = jax.lax.broadcasted_iota(jnp.int32, (T, T), 0)
        tj = jax.lax.broadcasted_iota(jnp.int32, (T, T), 1)
        mask = (ti // S == tj // S) & (tj % S <= ti % S)
        neg = jnp.float32(-1e30)
        scale = jnp.float32(1.0 / math.sqrt(DH))

        outs = []
        for hh in range(H):
            j = hh // (H // HK)
            qh = rope(_rms(q2[:, hh * DH:(hh + 1) * DH], qn))
            sc = jax.lax.dot_general(qh, ks[j], (((1,), (1,)), ((), ())),
                                     preferred_element_type=jnp.float32)
            sc = jnp.where(mask, sc * scale, neg)
            ex = jnp.exp(sc - jnp.max(sc, axis=1, keepdims=True))
            at = ex / jnp.sum(ex, axis=1, keepdims=True)
            outs.append(jnp.dot(at, vs[j], preferred_element_type=jnp.float32))
        o2 = jnp.concatenate(outs, axis=1)
        hidden = h + jnp.dot(o2, wo_ref[...],
                             preferred_element_type=jnp.float32)
        out_ref[...] = hidden.reshape(B, S, D)

        x2 = _rms(hidden, ln2_ref[...])
        logits = jnp.dot(x2, rw_ref[...], preferred_element_type=jnp.float32)

        eidx = jax.lax.broadcasted_iota(jnp.int32, (T, E), 1)
        rowmax = jnp.max(logits, axis=1, keepdims=True)
        assign = jnp.min(jnp.where(logits == rowmax, eidx, E), axis=1,
                         keepdims=True)
        onehot = (eidx == assign).astype(jnp.float32)
        counts = jnp.sum(onehot, axis=0, keepdims=True)
        padded = jnp.float32(8.0) * jnp.ceil(counts * jnp.float32(1.0 / 8.0))
        er = jax.lax.broadcasted_iota(jnp.int32, (E, E), 0)
        ec = jax.lax.broadcasted_iota(jnp.int32, (E, E), 1)
        lt_e = (er < ec).astype(jnp.float32)
        offsets = jnp.dot(padded, lt_e,
                          preferred_element_type=jnp.float32)
        lt_t = (tj < ti).astype(jnp.float32)
        intra = jnp.dot(lt_t, onehot,
                        preferred_element_type=jnp.float32)
        rank = jnp.sum(onehot * (offsets + intra), axis=1,
                       keepdims=True)
        rj = jax.lax.broadcasted_iota(jnp.int32, (T, TP), 1)
        p_mat = (rank.astype(jnp.int32) == rj).astype(jnp.float32)
        p_scr[...] = p_mat
        x2_scr[...] = jnp.dot(p_mat.T, x2, preferred_element_type=jnp.float32)
        se = jax.lax.broadcasted_iota(jnp.int32, (E, E * 128), 0)
        sl = jax.lax.broadcasted_iota(jnp.int32, (E, E * 128), 1)
        spread = (sl // 128 == se).astype(jnp.float32)
        off_scr[...] = jnp.dot(offsets * jnp.float32(1.0 / 8.0), spread,
                               preferred_element_type=jnp.float32
                               ).astype(jnp.int32)
        cnt_scr[...] = jnp.dot(counts, spread,
                               preferred_element_type=jnp.float32
                               ).astype(jnp.int32)

    @pl.when(i > 0)
    def _expert():
        e = i - 1
        slot = jax.lax.rem(e, NBUF)
        off8 = off_scr[0, e * 128]
        cnt = cnt_scr[0, e * 128]
        for half in range(2):
            _gu_copy(e, slot, half).wait()
            _d_copy(e, slot, half).wait()

        ntiles = (cnt + TILE - 1) // TILE
        wgu = gu_buf[slot]
        wd = d_buf[slot]
        rid = jax.lax.broadcasted_iota(jnp.int32, (TILE, 1), 0)

        def tile_step(t, _):
            base = off8 * 8 + t * TILE
            xb = x2_scr[pl.ds(base, TILE), :]
            gu = jnp.dot(xb, wgu, preferred_element_type=jnp.float32)
            gate = gu[:, :F]
            up = gu[:, F:]
            act = gate * jax.lax.logistic(gate) * up
            y = jnp.dot(act, wd, preferred_element_type=jnp.float32)
            valid = t * TILE + rid < cnt
            cur = y_scr[pl.ds(base, TILE), :]
            y_scr[pl.ds(base, TILE), :] = jnp.where(valid, y, cur)
            return 0

        jax.lax.fori_loop(0, ntiles, tile_step, 0)

        @pl.when(e + NBUF < E)
        def _next():
            start_copy(e + NBUF, slot)

    @pl.when(i == E)
    def _finish():
        moe = jnp.dot(p_scr[...], y_scr[...],
                      preferred_element_type=jnp.float32)
        out_ref[...] = out_ref[...] + moe.reshape(B, S, D)


@jax.jit
def kernel(hidden_states, ln1_w, wq, wk, wv, q_norm_w, k_norm_w, wo, ln2_w,
           router_w, w_gate_up, w_down):
    whole = lambda e: (0,) * 2
    out = pl.pallas_call(
        _fused_body,
        grid=(E + 1,),
        in_specs=[
            pl.BlockSpec((B, S, D), lambda e: (0, 0, 0)),
            pl.BlockSpec((1, D), whole),
            pl.BlockSpec((D, H * DH), whole),
            pl.BlockSpec((D, HK * DH), whole),
            pl.BlockSpec((D, HK * DH), whole),
            pl.BlockSpec((1, DH), whole),
            pl.BlockSpec((1, DH), whole),
            pl.BlockSpec((H * DH, D), whole),
            pl.BlockSpec((1, D), whole),
            pl.BlockSpec((D, E), whole),
            pl.BlockSpec(memory_space=pltpu.MemorySpace.HBM),
            pl.BlockSpec(memory_space=pltpu.MemorySpace.HBM),
        ],
        out_specs=pl.BlockSpec((B, S, D), lambda e: (0, 0, 0)),
        scratch_shapes=[
            pltpu.VMEM((TP, D), jnp.float32),
            pltpu.VMEM((T, TP), jnp.float32),
            pltpu.VMEM((1, E * 128), jnp.int32),
            pltpu.VMEM((1, E * 128), jnp.int32),
            pltpu.VMEM((TP, D), jnp.float32),
            pltpu.VMEM((NBUF, D, 2 * F), jnp.float32),
            pltpu.VMEM((NBUF, F, D), jnp.float32),
            pltpu.SemaphoreType.DMA((NBUF, 2)),
            pltpu.SemaphoreType.DMA((NBUF, 2)),
        ],
        out_shape=jax.ShapeDtypeStruct((B, S, D), jnp.float32),
    )(hidden_states, ln1_w.reshape(1, D), wq, wk, wv,
      q_norm_w.reshape(1, DH), k_norm_w.reshape(1, DH), wo,
      ln2_w.reshape(1, D), router_w, w_gate_up, w_down)
    return out
